# Initial kernel scaffold; baseline (speedup 1.0000x reference)
import functools

import jax
import jax.numpy as jnp
from jax import lax
from jax.experimental import pallas as pl
from jax.experimental.pallas import tpu as pltpu
from jax.experimental.pallas import tpu_sc as plsc

NB = 2
N = 2048
NN = NB * N
KNN = 16
RB = 256
AB = 128


def _ft_body(x_ref, m_ref, w_ref, b_ref, o_ref):
    xm = x_ref[...] * m_ref[...]
    o_ref[...] = jnp.dot(xm, w_ref[0], preferred_element_type=jnp.float32) + b_ref[...]


def _feature_transform(x, ft_mask, W_ft, b_ft):
    f_in = x.shape[1]
    dout = W_ft.shape[2]
    return pl.pallas_call(
        _ft_body,
        grid=(NB,),
        in_specs=[
            pl.BlockSpec((N, f_in), lambda b: (0, 0)),
            pl.BlockSpec((1, f_in), lambda b: (b, 0)),
            pl.BlockSpec((1, f_in, dout), lambda b: (b, 0, 0)),
            pl.BlockSpec((1, dout), lambda b: (b, 0)),
        ],
        out_specs=pl.BlockSpec((N, dout), lambda b: (b, 0)),
        out_shape=jax.ShapeDtypeStruct((NN, dout), jnp.float32),
    )(x, ft_mask, W_ft, b_ft)


def _proj_topk_body(hT_ref, hr_ref, wl_ref, wr_ref, gl_ref, gr_ref, idx_ref):
    b = pl.program_id(0)
    xr = hr_ref[...]
    gl_ref[...] = jnp.dot(xr, wl_ref[...], preferred_element_type=jnp.float32)
    gr_ref[...] = jnp.dot(xr, wr_ref[...], preferred_element_type=jnp.float32)
    hT = hT_ref[...]
    sq = jnp.sum(hT * hT, axis=0, keepdims=True)
    d = sq - 2.0 * jnp.dot(xr, hT, preferred_element_type=jnp.float32)
    iota = lax.broadcasted_iota(jnp.int32, (RB, N), 1)
    cols = []
    for _ in range(KNN):
        m = jnp.min(d, axis=1, keepdims=True)
        am = jnp.min(jnp.where(d <= m, iota, N), axis=1, keepdims=True)
        cols.append(am)
        d = jnp.where(iota == am, jnp.inf, d)
    idx_ref[...] = jnp.concatenate(cols, axis=1) + b * N


def _proj_topk(h, hT, Wl, Wr):
    d_in = h.shape[1]
    dout = Wl.shape[1]
    nrb = N // RB
    return pl.pallas_call(
        _proj_topk_body,
        grid=(NB, nrb),
        in_specs=[
            pl.BlockSpec((d_in, N), lambda b, r: (0, b)),
            pl.BlockSpec((RB, d_in), lambda b, r: (b * nrb + r, 0)),
            pl.BlockSpec((d_in, dout), lambda b, r: (0, 0)),
            pl.BlockSpec((d_in, dout), lambda b, r: (0, 0)),
        ],
        out_specs=[
            pl.BlockSpec((RB, dout), lambda b, r: (b * nrb + r, 0)),
            pl.BlockSpec((RB, dout), lambda b, r: (b * nrb + r, 0)),
            pl.BlockSpec((RB, KNN), lambda b, r: (b * nrb + r, 0)),
        ],
        out_shape=[
            jax.ShapeDtypeStruct((NN, dout), jnp.float32),
            jax.ShapeDtypeStruct((NN, dout), jnp.float32),
            jax.ShapeDtypeStruct((NN, KNN), jnp.int32),
        ],
    )(hT, h, Wl, Wr)


def _sc_gather(table, idx):
    dout = table.shape[1]
    B = idx.shape[0]
    NW = 32
    b_per_w = B // NW
    R = 128
    n_chunks = b_per_w // R
    mesh = plsc.VectorSubcoreMesh(core_axis_name="c", subcore_axis_name="s")

    @functools.partial(
        pl.kernel,
        out_type=jax.ShapeDtypeStruct((B, dout), jnp.float32),
        mesh=mesh,
        scratch_types=[
            pltpu.VMEM((R,), jnp.int32),
            pltpu.VMEM((R, dout), jnp.float32),
            pltpu.SemaphoreType.DMA,
        ],
    )
    def k(table_hbm, idx_hbm, out_hbm, idx_v, rows_v, sem):
        wid = lax.axis_index("s") * 2 + lax.axis_index("c")
        base = wid * b_per_w

        def body(c, carry):
            off = base + c * R
            pltpu.sync_copy(idx_hbm.at[pl.ds(off, R)], idx_v)
            pltpu.async_copy(table_hbm.at[idx_v], rows_v, sem).wait()
            pltpu.sync_copy(rows_v, out_hbm.at[pl.ds(off, R)])
            return carry

        lax.fori_loop(0, n_chunks, body, 0)

    return k(table, idx)


def _attn_body(gl_ref, nbr_ref, a_ref, o_ref):
    gl = gl_ref[...]
    nbr = nbr_ref[...]
    z = gl[:, None, :] + nbr
    z = jnp.where(z >= 0.0, z, 0.2 * z)
    e = jnp.sum(z * a_ref[...][None, :, :], axis=2)
    e = e - jnp.max(e, axis=1, keepdims=True)
    w = jnp.exp(e)
    w = w / jnp.sum(w, axis=1, keepdims=True)
    o_ref[...] = jnp.sum(w[:, :, None] * nbr, axis=1)


def _attention(gl, nbr3, a2):
    dout = gl.shape[1]
    nab = NN // AB
    return pl.pallas_call(
        _attn_body,
        grid=(nab,),
        in_specs=[
            pl.BlockSpec((AB, dout), lambda i: (i, 0)),
            pl.BlockSpec((AB, KNN, dout), lambda i: (i, 0, 0)),
            pl.BlockSpec((1, dout), lambda i: (0, 0)),
        ],
        out_specs=pl.BlockSpec((AB, dout), lambda i: (i, 0)),
        out_shape=jax.ShapeDtypeStruct((NN, dout), jnp.float32),
    )(gl, nbr3, a2)


def _gat_layer(h, Wl, Wr, a):
    gl, gr, idx = _proj_topk(h, h.T, Wl, Wr)
    nbr = _sc_gather(gr, idx.reshape(-1))
    return _attention(gl, nbr.reshape(NN, KNN, -1), a[None, :])


def _mlp_body(m_ref, w1_ref, b1_ref, w2_ref, b2_ref, wg_ref, bg_ref, gv_ref, gs_ref, o_ref):
    h = jnp.dot(m_ref[...], w1_ref[...], preferred_element_type=jnp.float32) + b1_ref[...]
    h = jnp.maximum(h, 0.0)
    h = jnp.dot(h, w2_ref[...], preferred_element_type=jnp.float32) + b2_ref[...]
    h = jnp.maximum(h, 0.0)
    g = jnp.dot(h, wg_ref[...], preferred_element_type=jnp.float32) + bg_ref[...]
    o_ref[...] = jnp.tanh(gs_ref[0] * g) * gv_ref[...]


def _mlp(merged, W1, b1, W2, b2, Wg, bg, geod_v, geod_scale):
    return pl.pallas_call(
        _mlp_body,
        in_specs=[
            pl.BlockSpec(merged.shape, lambda: (0, 0)),
            pl.BlockSpec(W1.shape, lambda: (0, 0)),
            pl.BlockSpec((1, b1.shape[0]), lambda: (0, 0)),
            pl.BlockSpec(W2.shape, lambda: (0, 0)),
            pl.BlockSpec((1, b2.shape[0]), lambda: (0, 0)),
            pl.BlockSpec(Wg.shape, lambda: (0, 0)),
            pl.BlockSpec((1, bg.shape[0]), lambda: (0, 0)),
            pl.BlockSpec((N, 1), lambda: (0, 0)),
            pl.BlockSpec(memory_space=pltpu.SMEM),
        ],
        out_specs=pl.BlockSpec((N, 3), lambda: (0, 0)),
        out_shape=jax.ShapeDtypeStruct((N, 3), jnp.float32),
    )(merged, W1, b1[None, :], W2, b2[None, :], Wg, bg[None, :],
      geod_v[:, None], geod_scale[None])


def kernel(x, ft_mask, W_ft, b_ft, Wl1, Wr1, a1, Wl2, Wr2, a2, Wl3, Wr3, a3,
           Wl4, Wr4, a4, W1, b1, W2, b2, Wg, bg, geod_v, geod_scale):
    h0 = _feature_transform(x, ft_mask, W_ft, b_ft)
    o1 = _gat_layer(h0, Wl1, Wr1, a1)
    o2 = _gat_layer(jnp.concatenate([h0, o1], axis=1), Wl2, Wr2, a2)
    o3 = _gat_layer(jnp.concatenate([o1, o2], axis=1), Wl3, Wr3, a3)
    o4 = _gat_layer(jnp.concatenate([o2, o3], axis=1), Wl4, Wr4, a4)
    merged = jnp.concatenate([o4[:N], o4[N:]], axis=1)
    return _mlp(merged, W1, b1, W2, b2, Wg, bg, geod_v, geod_scale)


# speedup vs baseline: 8.1794x; 8.1794x over previous
import functools

import jax
import jax.numpy as jnp
from jax import lax
from jax.experimental import pallas as pl
from jax.experimental.pallas import tpu as pltpu
from jax.experimental.pallas import tpu_sc as plsc

NB = 2
N = 2048
NN = NB * N
KNN = 16
RB = 256
AB = 128


def _ft_body(x_ref, m_ref, w_ref, b_ref, o_ref):
    xm = x_ref[...] * m_ref[0]
    o_ref[...] = jnp.dot(xm, w_ref[0], preferred_element_type=jnp.float32) + b_ref[0]


def _feature_transform(x, ft_mask, W_ft, b_ft):
    f_in = x.shape[1]
    dout = W_ft.shape[2]
    return pl.pallas_call(
        _ft_body,
        grid=(NB,),
        in_specs=[
            pl.BlockSpec((N, f_in), lambda b: (0, 0)),
            pl.BlockSpec((1, 1, f_in), lambda b: (b, 0, 0)),
            pl.BlockSpec((1, f_in, dout), lambda b: (b, 0, 0)),
            pl.BlockSpec((1, 1, dout), lambda b: (b, 0, 0)),
        ],
        out_specs=pl.BlockSpec((N, dout), lambda b: (b, 0)),
        out_shape=jax.ShapeDtypeStruct((NN, dout), jnp.float32),
    )(x, ft_mask[:, None, :], W_ft, b_ft[:, None, :])


def _proj_topk_body(hT_ref, hr_ref, sqr_ref, sqc_ref, wl_ref, wr_ref,
                    gl_ref, gr_ref, idx_ref):
    b = pl.program_id(0)
    xr = hr_ref[...]
    gl_ref[...] = jnp.dot(xr, wl_ref[...], preferred_element_type=jnp.float32)
    gr_ref[...] = jnp.dot(xr, wr_ref[...], preferred_element_type=jnp.float32)
    hT = hT_ref[...]
    xy = jnp.dot(xr, hT, preferred_element_type=jnp.float32)
    d = (sqr_ref[...] + sqc_ref[0]) - 2.0 * xy
    iota = lax.broadcasted_iota(jnp.int32, (RB, N), 1)
    cols = []
    for _ in range(KNN):
        m = jnp.min(d, axis=1, keepdims=True)
        am = jnp.min(jnp.where(d <= m, iota, N), axis=1, keepdims=True)
        cols.append(am)
        d = jnp.where(iota == am, jnp.inf, d)
    idx_ref[...] = jnp.concatenate(cols, axis=1) + b * N


def _proj_topk(h, hT, sq, Wl, Wr):
    d_in = h.shape[1]
    dout = Wl.shape[1]
    nrb = N // RB
    return pl.pallas_call(
        _proj_topk_body,
        grid=(NB, nrb),
        in_specs=[
            pl.BlockSpec((d_in, N), lambda b, r: (0, b)),
            pl.BlockSpec((RB, d_in), lambda b, r: (b * nrb + r, 0)),
            pl.BlockSpec((RB, 1), lambda b, r: (b * nrb + r, 0)),
            pl.BlockSpec((1, 1, N), lambda b, r: (b, 0, 0)),
            pl.BlockSpec((d_in, dout), lambda b, r: (0, 0)),
            pl.BlockSpec((d_in, dout), lambda b, r: (0, 0)),
        ],
        out_specs=[
            pl.BlockSpec((RB, dout), lambda b, r: (b * nrb + r, 0)),
            pl.BlockSpec((RB, dout), lambda b, r: (b * nrb + r, 0)),
            pl.BlockSpec((RB, KNN), lambda b, r: (b * nrb + r, 0)),
        ],
        out_shape=[
            jax.ShapeDtypeStruct((NN, dout), jnp.float32),
            jax.ShapeDtypeStruct((NN, dout), jnp.float32),
            jax.ShapeDtypeStruct((NN, KNN), jnp.int32),
        ],
    )(hT, h, sq.reshape(NN, 1), sq.reshape(NB, 1, N), Wl, Wr)


def _sc_gather(table, idx):
    dout = table.shape[1]
    B = idx.shape[0]
    NW = 32
    b_per_w = B // NW
    R = 128
    n_chunks = b_per_w // R
    mesh = plsc.VectorSubcoreMesh(core_axis_name="c", subcore_axis_name="s")

    @functools.partial(
        pl.kernel,
        out_type=jax.ShapeDtypeStruct((B, dout), jnp.float32),
        mesh=mesh,
        scratch_types=[
            pltpu.VMEM((R,), jnp.int32),
            pltpu.VMEM((R, dout), jnp.float32),
            pltpu.SemaphoreType.DMA,
        ],
    )
    def k(table_hbm, idx_hbm, out_hbm, idx_v, rows_v, sem):
        wid = lax.axis_index("s") * 2 + lax.axis_index("c")
        base = wid * b_per_w

        def body(c, carry):
            off = base + c * R
            pltpu.sync_copy(idx_hbm.at[pl.ds(off, R)], idx_v)
            pltpu.async_copy(table_hbm.at[idx_v], rows_v, sem).wait()
            pltpu.sync_copy(rows_v, out_hbm.at[pl.ds(off, R)])
            return carry

        lax.fori_loop(0, n_chunks, body, 0)

    return k(table, idx)


def _attn_body(gl_ref, nbr_ref, a_ref, o_ref):
    gl = gl_ref[...]
    nbr = nbr_ref[...]
    dout = gl.shape[1]
    z = gl[:, None, :] + nbr
    z = jnp.where(z >= 0.0, z, 0.2 * z)
    z2 = z.reshape(AB * KNN, dout)
    e = jnp.dot(z2.astype(jnp.bfloat16), a_ref[...].astype(jnp.bfloat16),
                preferred_element_type=jnp.float32).reshape(AB, KNN)
    w = jnp.exp(e - jnp.max(e, axis=1, keepdims=True))
    s = w[:, :8] + w[:, 8:]
    s = s[:, :4] + s[:, 4:]
    s = s[:, :2] + s[:, 2:]
    s = s[:, :1] + s[:, 1:]
    w = w / s
    o_ref[...] = jnp.sum(w[:, :, None] * nbr, axis=1)


def _attention(gl, nbr3, a_col):
    dout = gl.shape[1]
    nab = NN // AB
    return pl.pallas_call(
        _attn_body,
        grid=(nab,),
        in_specs=[
            pl.BlockSpec((AB, dout), lambda i: (i, 0)),
            pl.BlockSpec((AB, KNN, dout), lambda i: (i, 0, 0)),
            pl.BlockSpec((dout, 1), lambda i: (0, 0)),
        ],
        out_specs=pl.BlockSpec((AB, dout), lambda i: (i, 0)),
        out_shape=jax.ShapeDtypeStruct((NN, dout), jnp.float32),
    )(gl, nbr3, a_col)


def _gat_layer(h, Wl, Wr, a):
    xb = h.reshape(NB, N, h.shape[1])
    sq = jnp.sum(xb * xb, axis=-1)
    gl, gr, idx = _proj_topk(h, h.T, sq, Wl, Wr)
    nbr = _sc_gather(gr, idx.reshape(-1))
    return _attention(gl, nbr.reshape(NN, KNN, -1), a[:, None])


def _mlp_body(m_ref, w1_ref, b1_ref, w2_ref, b2_ref, wg_ref, bg_ref, gv_ref, gs_ref, o_ref):
    h = jnp.dot(m_ref[...], w1_ref[...], preferred_element_type=jnp.float32) + b1_ref[...]
    h = jnp.maximum(h, 0.0)
    h = jnp.dot(h, w2_ref[...], preferred_element_type=jnp.float32) + b2_ref[...]
    h = jnp.maximum(h, 0.0)
    g = jnp.dot(h, wg_ref[...], preferred_element_type=jnp.float32) + bg_ref[...]
    o_ref[...] = jnp.tanh(gs_ref[0] * g) * gv_ref[...]


def _mlp(merged, W1, b1, W2, b2, Wg, bg, geod_v, geod_scale):
    return pl.pallas_call(
        _mlp_body,
        in_specs=[
            pl.BlockSpec(merged.shape, lambda: (0, 0)),
            pl.BlockSpec(W1.shape, lambda: (0, 0)),
            pl.BlockSpec((1, b1.shape[0]), lambda: (0, 0)),
            pl.BlockSpec(W2.shape, lambda: (0, 0)),
            pl.BlockSpec((1, b2.shape[0]), lambda: (0, 0)),
            pl.BlockSpec(Wg.shape, lambda: (0, 0)),
            pl.BlockSpec((1, bg.shape[0]), lambda: (0, 0)),
            pl.BlockSpec((N, 1), lambda: (0, 0)),
            pl.BlockSpec(memory_space=pltpu.SMEM),
        ],
        out_specs=pl.BlockSpec((N, 3), lambda: (0, 0)),
        out_shape=jax.ShapeDtypeStruct((N, 3), jnp.float32),
    )(merged, W1, b1[None, :], W2, b2[None, :], Wg, bg[None, :],
      geod_v[:, None], geod_scale[None])


def kernel(x, ft_mask, W_ft, b_ft, Wl1, Wr1, a1, Wl2, Wr2, a2, Wl3, Wr3, a3,
           Wl4, Wr4, a4, W1, b1, W2, b2, Wg, bg, geod_v, geod_scale):
    h0 = _feature_transform(x, ft_mask, W_ft, b_ft)
    o1 = _gat_layer(h0, Wl1, Wr1, a1)
    o2 = _gat_layer(jnp.concatenate([h0, o1], axis=1), Wl2, Wr2, a2)
    o3 = _gat_layer(jnp.concatenate([o1, o2], axis=1), Wl3, Wr3, a3)
    o4 = _gat_layer(jnp.concatenate([o2, o3], axis=1), Wl4, Wr4, a4)
    merged = jnp.concatenate([o4[:N], o4[N:]], axis=1)
    return _mlp(merged, W1, b1, W2, b2, Wg, bg, geod_v, geod_scale)
